# baseline (speedup 1.0000x reference)
import functools

import jax
import jax.numpy as jnp
from jax import lax
from jax.experimental import pallas as pl
from jax.experimental.pallas import tpu as pltpu
from jax.experimental.pallas import tpu_sc as plsc

D_MODEL = 64
SCALE = float(D_MODEL) ** 0.5
NC = 2
NS = 16
NW = NC * NS
L = 16

CHUNK = 400
NBUF = 2


@functools.cache
def _build(B: int):
    b_per_w = B // NW
    n_chunks = b_per_w // CHUNK
    n_groups = n_chunks // NBUF
    assert n_chunks * CHUNK == b_per_w and n_groups * NBUF == n_chunks
    assert n_groups >= 2
    mesh = plsc.VectorSubcoreMesh(core_axis_name="c", subcore_axis_name="s")

    @functools.partial(
        pl.kernel,
        mesh=mesh,
        out_type=jax.ShapeDtypeStruct((B, D_MODEL), jnp.float32),
        scratch_types=[
            pltpu.VMEM((NBUF, CHUNK), jnp.int32),
            pltpu.VMEM((NBUF, CHUNK, D_MODEL), jnp.float32),
            pltpu.VMEM((NBUF, CHUNK, D_MODEL), jnp.float32),
            pltpu.SemaphoreType.DMA,
            pltpu.SemaphoreType.DMA,
            pltpu.SemaphoreType.DMA,
            pltpu.SemaphoreType.DMA,
        ],
        compiler_params=pltpu.CompilerParams(use_tc_tiling_on_sc=False),
    )
    def emb(tokens_hbm, table_hbm, out_hbm, idx_v, rg_v, rs_v,
            gsem0, gsem1, ssem0, ssem1):
        gsems = (gsem0, gsem1)
        ssems = (ssem0, ssem1)
        wid = lax.axis_index("s") * NC + lax.axis_index("c")
        base = wid * b_per_w

        def fire_gather(ci, b):
            start = base + ci * CHUNK
            pltpu.sync_copy(tokens_hbm.at[pl.ds(start, CHUNK)], idx_v.at[b])
            pltpu.async_copy(table_hbm.at[idx_v.at[b]], rg_v.at[b], gsems[b])

        def wait_gather(b):
            pltpu.make_async_copy(table_hbm.at[idx_v.at[b]], rg_v.at[b],
                                  gsems[b]).wait()

        def fire_store(ci, b):
            start = base + ci * CHUNK
            pltpu.async_copy(rs_v.at[b], out_hbm.at[pl.ds(start, CHUNK)],
                             ssems[b])

        def wait_store(b):
            pltpu.make_async_copy(rs_v.at[b], out_hbm.at[pl.ds(base, CHUNK)],
                                  ssems[b]).wait()

        def scale_chunk(b):
            def row_body(r, acc):
                for c in range(D_MODEL // L):
                    sl = pl.ds(c * L, L)
                    rs_v[b, r, sl] = rg_v[b, r, sl] * SCALE
                return acc

            lax.fori_loop(0, CHUNK, row_body, 0, unroll=4)

        for b in range(NBUF):
            fire_gather(b, b)

        for b in range(NBUF):
            wait_gather(b)
            scale_chunk(b)
            fire_store(b, b)
            fire_gather(NBUF + b, b)

        def group_body(gi, carry):
            for b in range(NBUF):
                ci = gi * NBUF + b
                wait_gather(b)
                wait_store(b)
                scale_chunk(b)
                fire_store(ci, b)
                fire_gather(ci + NBUF, b)
            return carry

        lax.fori_loop(1, n_groups - 1, group_body, 0, unroll=False)

        for b in range(NBUF):
            ci = (n_groups - 1) * NBUF + b
            wait_gather(b)
            wait_store(b)
            scale_chunk(b)
            fire_store(ci, b)
        for b in range(NBUF):
            wait_store(b)

    return emb


def kernel(tokens, table):
    b, s = tokens.shape
    flat = b * s
    idx = tokens.reshape(flat).astype(jnp.int32)
    out = _build(flat)(idx, table)
    return out.reshape(b, s, D_MODEL)
